# baseline (speedup 1.0000x reference)
import functools

import jax
import jax.numpy as jnp
from jax import lax
from jax.experimental import pallas as pl
from jax.experimental.pallas import tpu as pltpu
from jax.experimental.pallas import tpu_sc as plsc

_NUM_CORES = 2
_NUM_SUBCORES = 16
_NUM_WORKERS = _NUM_CORES * _NUM_SUBCORES

_CHUNK = 256
_NSEM = 4


def _sc_gather(label, cls_ctx):
    b = label.shape[0]
    n_ctx, d = cls_ctx.shape[1], cls_ctx.shape[2]
    bpw = b // _NUM_WORKERS

    mesh = plsc.VectorSubcoreMesh(core_axis_name="c", subcore_axis_name="s")

    @functools.partial(
        pl.kernel,
        mesh=mesh,
        out_type=jax.ShapeDtypeStruct((b, n_ctx, d), jnp.float32),
        scratch_types=[
            pltpu.VMEM((bpw,), jnp.int32),
            pltpu.VMEM((bpw, n_ctx, d), jnp.float32),
            pltpu.SemaphoreType.DMA,
        ],
    )
    def body(label_hbm, table_hbm, out_hbm, idx_v, rows_v, sem):
        wid = lax.axis_index("s") * _NUM_CORES + lax.axis_index("c")
        base = wid * bpw
        pltpu.sync_copy(label_hbm.at[pl.ds(base, bpw)], idx_v)
        pltpu.async_copy(table_hbm.at[idx_v], rows_v, sem).wait()
        pltpu.sync_copy(rows_v, out_hbm.at[pl.ds(base, bpw)])

    return body(label, cls_ctx)


def _tc_fill(cls, token_prefix, token_suffix):
    b, n_ctx, d = cls.shape
    pre = token_prefix.shape[1]
    suf = token_suffix.shape[1]
    tok = pre + n_ctx + suf
    n_chunks = b // _CHUNK

    def body(cls_ref, pre_ref, suf_ref, out_ref, pre_rep, suf_rep, *sems):
        pre_rep[:] = jnp.broadcast_to(pre_ref[:], (pre, _CHUNK, d))
        suf_rep[:] = jnp.broadcast_to(
            jnp.transpose(suf_ref[:], (1, 0, 2)), (suf, _CHUNK, d))

        copies = []
        for t in range(n_ctx):
            copies.append(pltpu.make_async_copy(
                cls_ref.at[:, t, :], out_ref.at[pre + t],
                sems[t % _NSEM]))
        for c in range(n_chunks):
            b0 = c * _CHUNK
            copies.append(pltpu.make_async_copy(
                pre_rep, out_ref.at[pl.ds(0, pre), pl.ds(b0, _CHUNK), :],
                sems[c % _NSEM]))
            for t in range(suf):
                copies.append(pltpu.make_async_copy(
                    suf_rep.at[t],
                    out_ref.at[pre + n_ctx + t, pl.ds(b0, _CHUNK), :],
                    sems[(c + t) % _NSEM]))
        for cp in copies:
            cp.start()
        for cp in copies:
            cp.wait()

    return pl.pallas_call(
        body,
        in_specs=[
            pl.BlockSpec(memory_space=pl.ANY),
            pl.BlockSpec(memory_space=pltpu.VMEM),
            pl.BlockSpec(memory_space=pltpu.VMEM),
        ],
        out_specs=pl.BlockSpec(memory_space=pl.ANY),
        out_shape=jax.ShapeDtypeStruct((tok, b, d), jnp.float32),
        scratch_shapes=[
            pltpu.VMEM((pre, _CHUNK, d), jnp.float32),
            pltpu.VMEM((suf, _CHUNK, d), jnp.float32),
        ] + [pltpu.SemaphoreType.DMA] * _NSEM,
    )(cls, token_prefix, token_suffix)


def kernel(label, cls_ctx, token_prefix, token_suffix):
    cls = _sc_gather(label, cls_ctx)
    out_t = _tc_fill(cls, token_prefix, token_suffix)
    return jnp.transpose(out_t, (1, 0, 2))


# speedup vs baseline: 3.5445x; 3.5445x over previous
import functools

import jax
import jax.numpy as jnp
from jax import lax
from jax.experimental import pallas as pl
from jax.experimental.pallas import tpu as pltpu
from jax.experimental.pallas import tpu_sc as plsc

_NUM_CORES = 2
_NUM_SUBCORES = 16
_NUM_WORKERS = _NUM_CORES * _NUM_SUBCORES

_TBLK = 8


def _sc_gather(label, cls_ctx):
    b = label.shape[0]
    n_ctx, d = cls_ctx.shape[1], cls_ctx.shape[2]
    bpw = b // _NUM_WORKERS

    mesh = plsc.VectorSubcoreMesh(core_axis_name="c", subcore_axis_name="s")

    @functools.partial(
        pl.kernel,
        mesh=mesh,
        out_type=jax.ShapeDtypeStruct((b, n_ctx, d), jnp.float32),
        scratch_types=[
            pltpu.VMEM((bpw,), jnp.int32),
            pltpu.VMEM((bpw, n_ctx, d), jnp.float32),
            pltpu.SemaphoreType.DMA,
        ],
    )
    def body(label_hbm, table_hbm, out_hbm, idx_v, rows_v, sem):
        wid = lax.axis_index("s") * _NUM_CORES + lax.axis_index("c")
        base = wid * bpw
        pltpu.sync_copy(label_hbm.at[pl.ds(base, bpw)], idx_v)
        pltpu.async_copy(table_hbm.at[idx_v], rows_v, sem).wait()
        pltpu.sync_copy(rows_v, out_hbm.at[pl.ds(base, bpw)])

    return body(label, cls_ctx)


def _tc_fill(cls, token_prefix, suffix_t):
    b, n_ctx, d = cls.shape
    pre = token_prefix.shape[1]
    suf = suffix_t.shape[0]
    tok = pre + n_ctx + suf
    lead = pre + n_ctx
    grid = (pl.cdiv(tok, _TBLK),)

    def body(cls_ref, pre_ref, suf_ref, out_ref):
        i = pl.program_id(0)

        @pl.when(i == 0)
        def _first():
            out_ref[0] = jnp.broadcast_to(pre_ref[0], (b, d))
            for t in range(1, lead):
                out_ref[t] = cls_ref[:, t - 1, :]
            for k in range(lead, _TBLK):
                out_ref[k] = jnp.broadcast_to(suf_ref[k - lead], (b, d))

        @pl.when(i > 0)
        def _rest():
            for k in range(_TBLK):
                out_ref[k] = jnp.broadcast_to(suf_ref[k], (b, d))

    return pl.pallas_call(
        body,
        grid=grid,
        in_specs=[
            pl.BlockSpec((b, n_ctx, d), lambda i: (0, 0, 0)),
            pl.BlockSpec((1, pre, d), lambda i: (0, 0, 0)),
            pl.BlockSpec((_TBLK, 1, d),
                         lambda i: (jnp.maximum(i * _TBLK - lead, 0), 0, 0)),
        ],
        out_specs=pl.BlockSpec((_TBLK, b, d), lambda i: (i, 0, 0)),
        out_shape=jax.ShapeDtypeStruct((tok, b, d), jnp.float32),
    )(cls, token_prefix, suffix_t)


def kernel(label, cls_ctx, token_prefix, token_suffix):
    cls = _sc_gather(label, cls_ctx)
    suffix_t = jnp.transpose(token_suffix, (1, 0, 2))
    out_t = _tc_fill(cls, token_prefix, suffix_t)
    return jnp.transpose(out_t, (1, 0, 2))


# speedup vs baseline: 3.5466x; 1.0006x over previous
import functools

import jax
import jax.numpy as jnp
from jax import lax
from jax.experimental import pallas as pl
from jax.experimental.pallas import tpu as pltpu
from jax.experimental.pallas import tpu_sc as plsc

_NUM_CORES = 2
_NUM_SUBCORES = 16
_NUM_WORKERS = _NUM_CORES * _NUM_SUBCORES

_TBLK = 7


def _sc_gather(label, cls_ctx):
    b = label.shape[0]
    n_ctx, d = cls_ctx.shape[1], cls_ctx.shape[2]
    bpw = b // _NUM_WORKERS

    mesh = plsc.VectorSubcoreMesh(core_axis_name="c", subcore_axis_name="s")

    @functools.partial(
        pl.kernel,
        mesh=mesh,
        out_type=jax.ShapeDtypeStruct((b, n_ctx, d), jnp.float32),
        scratch_types=[
            pltpu.VMEM((bpw,), jnp.int32),
            pltpu.VMEM((bpw, n_ctx, d), jnp.float32),
            pltpu.SemaphoreType.DMA,
        ],
    )
    def body(label_hbm, table_hbm, out_hbm, idx_v, rows_v, sem):
        wid = lax.axis_index("s") * _NUM_CORES + lax.axis_index("c")
        base = wid * bpw
        pltpu.sync_copy(label_hbm.at[pl.ds(base, bpw)], idx_v)
        pltpu.async_copy(table_hbm.at[idx_v], rows_v, sem).wait()
        pltpu.sync_copy(rows_v, out_hbm.at[pl.ds(base, bpw)])

    return body(label, cls_ctx)


def _tc_fill(cls, token_prefix, suffix_t):
    b, n_ctx, d = cls.shape
    pre = token_prefix.shape[1]
    suf = suffix_t.shape[0]
    tok = pre + n_ctx + suf
    lead = pre + n_ctx
    grid = (pl.cdiv(tok, _TBLK),)

    def body(cls_ref, pre_ref, suf_ref, out_ref):
        i = pl.program_id(0)

        @pl.when(i == 0)
        def _first():
            out_ref[0] = jnp.broadcast_to(pre_ref[0], (b, d))
            for t in range(1, lead):
                out_ref[t] = cls_ref[:, t - 1, :]
            for k in range(lead, _TBLK):
                out_ref[k] = jnp.broadcast_to(suf_ref[k - lead, 0], (b, d))

        @pl.when(i > 0)
        def _rest():
            for k in range(_TBLK):
                row = suf_ref[pl.ds(i * _TBLK + k - lead, 1)]
                out_ref[k] = jnp.broadcast_to(row[0], (b, d))

    return pl.pallas_call(
        body,
        grid=grid,
        in_specs=[
            pl.BlockSpec((b, n_ctx, d), lambda i: (0, 0, 0)),
            pl.BlockSpec((1, pre, d), lambda i: (0, 0, 0)),
            pl.BlockSpec((suf, 1, d), lambda i: (0, 0, 0)),
        ],
        out_specs=pl.BlockSpec((_TBLK, b, d), lambda i: (i, 0, 0)),
        out_shape=jax.ShapeDtypeStruct((tok, b, d), jnp.float32),
    )(cls, token_prefix, suffix_t)


def kernel(label, cls_ctx, token_prefix, token_suffix):
    cls = _sc_gather(label, cls_ctx)
    suffix_t = jnp.transpose(token_suffix, (1, 0, 2))
    out_t = _tc_fill(cls, token_prefix, suffix_t)
    return jnp.transpose(out_t, (1, 0, 2))
